# baseline (speedup 1.0000x reference)
import functools

import jax
import jax.numpy as jnp
from jax import lax
from jax.experimental import pallas as pl
from jax.experimental.pallas import tpu as pltpu
from jax.experimental.pallas import tpu_sc as plsc

_MARGIN = 0.2
_K = 32
_NEG_INF = -3.0e38


_TOL = 4096.0


def _bisect(keys, k, lo, hi, axis, n_iter=None, n_pre=12):
    def step(carry):
        lo, hi, c_lo = carry
        mid = lo + ((hi - lo + 1) >> 1)
        cnt = jnp.sum((keys >= mid).astype(jnp.int32), axis=axis, keepdims=True)
        ge = cnt >= k
        return (jnp.where(ge, mid, lo), jnp.where(ge, hi, mid - 1),
                jnp.where(ge, cnt, c_lo))

    c0 = jnp.full(lo.shape, keys.shape[axis], jnp.int32)
    carry = (lo, hi, c0)
    if n_iter is not None:
        for _ in range(n_iter):
            carry = step(carry)
        lo, hi, _ = carry
    else:
        def cond(c):
            lo, hi, c_lo = c
            width = (hi - lo).astype(jnp.float32)
            extra = (c_lo - k).astype(jnp.float32)
            return jnp.any(width * extra > _TOL)

        for _ in range(n_pre):
            carry = step(carry)
        lo, hi, _ = jax.lax.while_loop(cond, step, carry)
    return lo


def _topk_sum(keys, k, lo, hi, axis):
    t_bits = _bisect(keys, k, lo, hi, axis)
    t = jax.lax.bitcast_convert_type(t_bits, jnp.float32)
    gt = keys > t_bits
    vals = jax.lax.bitcast_convert_type(keys, jnp.float32)
    s = jnp.sum(jnp.where(gt, vals, 0.0), axis=axis, keepdims=True)
    c = jnp.sum(gt.astype(jnp.float32), axis=axis, keepdims=True)
    return s + (k.astype(jnp.float32) - c) * t


def _row_body(neg_ref, x_ref, o_ref, keys_ref, *, block: int):
    i = pl.program_id(0)
    R = block
    x = x_ref[...]
    xd = x_ref[:, pl.ds(i * R, R)]
    rr = jax.lax.broadcasted_iota(jnp.int32, (R, R), 0)
    cc = jax.lax.broadcasted_iota(jnp.int32, (R, R), 1)
    deq = rr == cc
    d = jnp.sum(jnp.where(deq, xd, 0.0), axis=1, keepdims=True)
    cost = jnp.maximum(x + (_MARGIN - d), 0.0)
    keys_ref[...] = jax.lax.bitcast_convert_type(cost, jnp.int32)
    dblk = keys_ref[:, pl.ds(i * R, R)]
    keys_ref[:, pl.ds(i * R, R)] = jnp.where(deq, 0, dblk)
    keys = keys_ref[...]
    k = neg_ref[0]

    m = keys
    w = m.shape[1]
    while w > 128:
        w //= 2
        m = jnp.maximum(m[:, :w], m[:, w:])
    rowmax = jnp.max(m, axis=1, keepdims=True)
    zero = jnp.zeros((R, 1), jnp.int32)
    tau = _bisect(m, k, zero, rowmax, axis=1, n_iter=10)
    o_ref[...] = _topk_sum(keys, k, tau, rowmax, axis=1)


def _col_body(neg_ref, x_ref, o_ref, keys_ref, *, block: int):
    j = pl.program_id(0)
    C = block
    x = x_ref[...]
    xd = x_ref[pl.ds(j * C, C), :]
    rr = jax.lax.broadcasted_iota(jnp.int32, (C, C), 0)
    cc = jax.lax.broadcasted_iota(jnp.int32, (C, C), 1)
    deq = rr == cc
    d = jnp.sum(jnp.where(deq, xd, 0.0), axis=0, keepdims=True)
    cost = jnp.maximum(x + (_MARGIN - d), 0.0)
    keys_ref[...] = jax.lax.bitcast_convert_type(cost, jnp.int32)
    dblk = keys_ref[pl.ds(j * C, C), :]
    keys_ref[pl.ds(j * C, C), :] = jnp.where(deq, 0, dblk)
    keys = keys_ref[...]
    k = neg_ref[0]

    m = keys
    w = m.shape[0]
    while w > 128:
        w //= 2
        m = jnp.maximum(m[:w, :], m[w:, :])
    colmax = jnp.max(m, axis=0, keepdims=True)
    zero = jnp.zeros((1, C), jnp.int32)
    tau = _bisect(m, k, zero, colmax, axis=0, n_iter=10)
    res = _topk_sum(keys, k, tau, colmax, axis=0)
    o_ref[...] = jnp.broadcast_to(res, o_ref.shape)


def _run(scores, neg, *, block: int = 256, interpret: bool = False):
    n = scores.shape[0]
    neg_arr = jnp.asarray(neg, jnp.int32).reshape(1)
    grid = (n // block,)

    row_out = pl.pallas_call(
        functools.partial(_row_body, block=block),
        grid=grid,
        in_specs=[
            pl.BlockSpec(memory_space=pltpu.SMEM),
            pl.BlockSpec((block, n), lambda i: (i, 0)),
        ],
        out_specs=pl.BlockSpec((block, 1), lambda i: (i, 0)),
        out_shape=jax.ShapeDtypeStruct((n, 1), jnp.float32),
        scratch_shapes=[pltpu.VMEM((block, n), jnp.int32)],
        interpret=interpret,
    )(neg_arr, scores)

    col_out = pl.pallas_call(
        functools.partial(_col_body, block=block),
        grid=grid,
        in_specs=[
            pl.BlockSpec(memory_space=pltpu.SMEM),
            pl.BlockSpec((n, block), lambda j: (0, j)),
        ],
        out_specs=pl.BlockSpec((8, block), lambda j: (0, j)),
        out_shape=jax.ShapeDtypeStruct((8, n), jnp.float32),
        scratch_shapes=[pltpu.VMEM((n, block), jnp.int32)],
        interpret=interpret,
    )(neg_arr, scores)

    return (row_out[:, 0] + col_out[0, :]) / neg


def _run_col_tc(scores, neg, *, block: int = 256):
    n = scores.shape[0]
    neg_arr = jnp.asarray(neg, jnp.int32).reshape(1)
    col_out = pl.pallas_call(
        functools.partial(_col_body, block=block),
        grid=(n // block,),
        in_specs=[
            pl.BlockSpec(memory_space=pltpu.SMEM),
            pl.BlockSpec((n, block), lambda j: (0, j)),
        ],
        out_specs=pl.BlockSpec((8, block), lambda j: (0, j)),
        out_shape=jax.ShapeDtypeStruct((8, n), jnp.float32),
        scratch_shapes=[pltpu.VMEM((n, block), jnp.int32)],
    )(neg_arr, scores)
    return col_out[0, :]


def _sc_row_pass(scores):
    n = scores.shape[0]
    nw = 32
    rows_per = n // nw
    mesh = plsc.VectorSubcoreMesh(core_axis_name="c", subcore_axis_name="s")

    @functools.partial(
        pl.kernel,
        mesh=mesh,
        out_type=jax.ShapeDtypeStruct((n,), jnp.float32),
        compiler_params=pltpu.CompilerParams(needs_layout_passes=False),
        scratch_types=[
            pltpu.VMEM((n,), jnp.float32),
            pltpu.VMEM((n,), jnp.float32),
            pltpu.VMEM((n + 32,), jnp.float32),
            pltpu.VMEM((rows_per,), jnp.float32),
            pltpu.VMEM((16,), jnp.int32),
        ],
    )
    def k_sc(x_hbm, o_hbm, row_v, cost_v, cand_v, out_v, wp_v):
        wid = lax.axis_index("s") * 2 + lax.axis_index("c")
        base = wid * rows_per
        iota = lax.iota(jnp.int32, 16)
        zero16f = jnp.zeros((16,), jnp.float32)
        zero16i = jnp.zeros((16,), jnp.int32)

        def row_body(rl, _):
            r = base + rl
            pltpu.sync_copy(x_hbm.at[r], row_v)
            cstart = (r >> 4) << 4
            lane_v = jnp.full((16,), r - cstart, jnp.int32)
            chunk = row_v[pl.ds(cstart, 16)]
            d_s = jnp.max(jnp.where(iota == lane_v, chunk, _NEG_INF))
            row_v[pl.ds(cstart, 16)] = jnp.where(iota == lane_v, _NEG_INF, chunk)
            md = jnp.full((16,), _MARGIN - d_s, jnp.float32)

            def pa(i, carry):
                m0, m1 = carry
                c0 = i * 32
                v0 = row_v[pl.ds(c0, 16)]
                t0 = jnp.maximum(v0 + md, 0.0)
                cost_v[pl.ds(c0, 16)] = t0
                v1 = row_v[pl.ds(c0 + 16, 16)]
                t1 = jnp.maximum(v1 + md, 0.0)
                cost_v[pl.ds(c0 + 16, 16)] = t1
                return jnp.maximum(m0, t0), jnp.maximum(m1, t1)

            m0, m1 = lax.fori_loop(0, n // 32, pa, (zero16f, zero16f))
            tau_s = jnp.min(jnp.minimum(m0, m1))
            rmax_s = jnp.max(jnp.maximum(m0, m1))
            tau_v = jnp.full((16,), tau_s, jnp.float32)
            wp_v[...] = zero16i

            def pb(g, _):
                c0 = g * 64
                v0 = cost_v[pl.ds(c0, 16)]
                v1 = cost_v[pl.ds(c0 + 16, 16)]
                v2 = cost_v[pl.ds(c0 + 32, 16)]
                v3 = cost_v[pl.ds(c0 + 48, 16)]
                m = jnp.maximum(jnp.maximum(v0, v1), jnp.maximum(v2, v3))

                @pl.when(jnp.max(m) > tau_s)
                def _():
                    for v in (v0, v1, v2, v3):
                        msk = v > tau_v
                        mi = msk.astype(jnp.int32)
                        pos = plsc.cumsum(mi) - mi
                        wp = wp_v[...]
                        plsc.store_scatter(cand_v, [wp + pos], v, mask=msk)
                        wp_v[...] = wp + plsc.all_reduce_population_count(msk)

                return 0

            lax.fori_loop(0, n // 64, pb, 0)

            wp = wp_v[...]
            ls = jnp.max(wp)
            plsc.store_scatter(cand_v, [wp + iota], zero16f)
            nch = (ls + 15) >> 4
            tau_bits = jnp.max(plsc.bitcast(tau_v, jnp.int32))
            rmax_bits = jnp.max(
                plsc.bitcast(jnp.full((16,), rmax_s, jnp.float32), jnp.int32))

            def bis(_, ch):
                lo, hi = ch
                mid = lo + ((hi - lo + 1) >> 1)
                midv = jnp.full((16,), mid, jnp.int32)

                def cp(c, acc):
                    v = cand_v[pl.ds(c * 16, 16)]
                    kb = plsc.bitcast(v, jnp.int32)
                    return acc + (kb >= midv).astype(jnp.int32)

                acc = lax.fori_loop(0, nch, cp, zero16i)
                cnt = jnp.sum(acc)
                ge = cnt >= _K
                return jnp.where(ge, mid, lo), jnp.where(ge, hi, mid - 1)

            lo, _hi = lax.fori_loop(0, 31, bis, (tau_bits, rmax_bits))
            t_bits = jnp.where(ls >= _K, lo, tau_bits)
            tb_v = jnp.full((16,), t_bits, jnp.int32)
            t_val = jnp.max(plsc.bitcast(tb_v, jnp.float32))

            def sp(c, carry):
                s, cc = carry
                v = cand_v[pl.ds(c * 16, 16)]
                kb = plsc.bitcast(v, jnp.int32)
                gt = kb > tb_v
                return s + jnp.where(gt, v, 0.0), cc + gt.astype(jnp.int32)

            s_acc, c_acc = lax.fori_loop(0, nch, sp, (zero16f, zero16i))
            s_row = jnp.sum(s_acc) + (
                (_K - jnp.sum(c_acc)).astype(jnp.float32) * t_val)
            plsc.store_scatter(
                out_v, [jnp.full((16,), rl, jnp.int32)],
                jnp.full((16,), s_row, jnp.float32), mask=iota == 0)
            return 0

        lax.fori_loop(0, rows_per, row_body, 0)
        pltpu.sync_copy(out_v, o_hbm.at[pl.ds(base, rows_per)])

    return k_sc(scores)


def kernel(scores, neg):
    row = _sc_row_pass(scores)
    col = _run_col_tc(scores, neg)
    return (row + col) / neg


# speedup vs baseline: 1.0842x; 1.0842x over previous
import functools

import jax
import jax.numpy as jnp
from jax import lax
from jax.experimental import pallas as pl
from jax.experimental.pallas import tpu as pltpu
from jax.experimental.pallas import tpu_sc as plsc

_MARGIN = 0.2
_K = 32
_NEG_INF = -3.0e38


_TOL = 4096.0


def _bisect(keys, k, lo, hi, axis, n_iter=None, n_pre=12):
    def step(carry):
        lo, hi, c_lo = carry
        mid = lo + ((hi - lo + 1) >> 1)
        cnt = jnp.sum((keys >= mid).astype(jnp.int32), axis=axis, keepdims=True)
        ge = cnt >= k
        return (jnp.where(ge, mid, lo), jnp.where(ge, hi, mid - 1),
                jnp.where(ge, cnt, c_lo))

    c0 = jnp.full(lo.shape, keys.shape[axis], jnp.int32)
    carry = (lo, hi, c0)
    if n_iter is not None:
        for _ in range(n_iter):
            carry = step(carry)
        lo, hi, _ = carry
    else:
        def cond(c):
            lo, hi, c_lo = c
            width = (hi - lo).astype(jnp.float32)
            extra = (c_lo - k).astype(jnp.float32)
            return jnp.any(width * extra > _TOL)

        for _ in range(n_pre):
            carry = step(carry)
        lo, hi, _ = jax.lax.while_loop(cond, step, carry)
    return lo


def _topk_sum(keys, k, lo, hi, axis):
    t_bits = _bisect(keys, k, lo, hi, axis)
    t = jax.lax.bitcast_convert_type(t_bits, jnp.float32)
    gt = keys > t_bits
    vals = jax.lax.bitcast_convert_type(keys, jnp.float32)
    s = jnp.sum(jnp.where(gt, vals, 0.0), axis=axis, keepdims=True)
    c = jnp.sum(gt.astype(jnp.float32), axis=axis, keepdims=True)
    return s + (k.astype(jnp.float32) - c) * t


def _row_body(neg_ref, x_ref, o_ref, keys_ref, *, block: int):
    i = pl.program_id(0)
    R = block
    x = x_ref[...]
    xd = x_ref[:, pl.ds(i * R, R)]
    rr = jax.lax.broadcasted_iota(jnp.int32, (R, R), 0)
    cc = jax.lax.broadcasted_iota(jnp.int32, (R, R), 1)
    deq = rr == cc
    d = jnp.sum(jnp.where(deq, xd, 0.0), axis=1, keepdims=True)
    cost = jnp.maximum(x + (_MARGIN - d), 0.0)
    keys_ref[...] = jax.lax.bitcast_convert_type(cost, jnp.int32)
    dblk = keys_ref[:, pl.ds(i * R, R)]
    keys_ref[:, pl.ds(i * R, R)] = jnp.where(deq, 0, dblk)
    keys = keys_ref[...]
    k = neg_ref[0]

    m = keys
    w = m.shape[1]
    while w > 128:
        w //= 2
        m = jnp.maximum(m[:, :w], m[:, w:])
    rowmax = jnp.max(m, axis=1, keepdims=True)
    zero = jnp.zeros((R, 1), jnp.int32)
    tau = _bisect(m, k, zero, rowmax, axis=1, n_iter=10)
    o_ref[...] = _topk_sum(keys, k, tau, rowmax, axis=1)


def _col_body(neg_ref, x_ref, o_ref, keys_ref, *, block: int):
    j = pl.program_id(0)
    C = block
    x = x_ref[...]
    xd = x_ref[pl.ds(j * C, C), :]
    rr = jax.lax.broadcasted_iota(jnp.int32, (C, C), 0)
    cc = jax.lax.broadcasted_iota(jnp.int32, (C, C), 1)
    deq = rr == cc
    d = jnp.sum(jnp.where(deq, xd, 0.0), axis=0, keepdims=True)
    cost = jnp.maximum(x + (_MARGIN - d), 0.0)
    keys_ref[...] = jax.lax.bitcast_convert_type(cost, jnp.int32)
    dblk = keys_ref[pl.ds(j * C, C), :]
    keys_ref[pl.ds(j * C, C), :] = jnp.where(deq, 0, dblk)
    keys = keys_ref[...]
    k = neg_ref[0]

    m = keys
    w = m.shape[0]
    while w > 128:
        w //= 2
        m = jnp.maximum(m[:w, :], m[w:, :])
    colmax = jnp.max(m, axis=0, keepdims=True)
    zero = jnp.zeros((1, C), jnp.int32)
    tau = _bisect(m, k, zero, colmax, axis=0, n_iter=10)
    res = _topk_sum(keys, k, tau, colmax, axis=0)
    o_ref[...] = jnp.broadcast_to(res, o_ref.shape)


def _run(scores, neg, *, block: int = 256, interpret: bool = False):
    n = scores.shape[0]
    neg_arr = jnp.asarray(neg, jnp.int32).reshape(1)
    grid = (n // block,)

    row_out = pl.pallas_call(
        functools.partial(_row_body, block=block),
        grid=grid,
        in_specs=[
            pl.BlockSpec(memory_space=pltpu.SMEM),
            pl.BlockSpec((block, n), lambda i: (i, 0)),
        ],
        out_specs=pl.BlockSpec((block, 1), lambda i: (i, 0)),
        out_shape=jax.ShapeDtypeStruct((n, 1), jnp.float32),
        scratch_shapes=[pltpu.VMEM((block, n), jnp.int32)],
        interpret=interpret,
    )(neg_arr, scores)

    col_out = pl.pallas_call(
        functools.partial(_col_body, block=block),
        grid=grid,
        in_specs=[
            pl.BlockSpec(memory_space=pltpu.SMEM),
            pl.BlockSpec((n, block), lambda j: (0, j)),
        ],
        out_specs=pl.BlockSpec((8, block), lambda j: (0, j)),
        out_shape=jax.ShapeDtypeStruct((8, n), jnp.float32),
        scratch_shapes=[pltpu.VMEM((n, block), jnp.int32)],
        interpret=interpret,
    )(neg_arr, scores)

    return (row_out[:, 0] + col_out[0, :]) / neg


def _run_col_tc(scores, neg, *, block: int = 256):
    n = scores.shape[0]
    neg_arr = jnp.asarray(neg, jnp.int32).reshape(1)
    col_out = pl.pallas_call(
        functools.partial(_col_body, block=block),
        grid=(n // block,),
        in_specs=[
            pl.BlockSpec(memory_space=pltpu.SMEM),
            pl.BlockSpec((n, block), lambda j: (0, j)),
        ],
        out_specs=pl.BlockSpec((8, block), lambda j: (0, j)),
        out_shape=jax.ShapeDtypeStruct((8, n), jnp.float32),
        scratch_shapes=[pltpu.VMEM((n, block), jnp.int32)],
    )(neg_arr, scores)
    return col_out[0, :]


def _sc_row_pass(scores):
    n = scores.shape[0]
    nw = 32
    rows_per = n // nw
    mesh = plsc.VectorSubcoreMesh(core_axis_name="c", subcore_axis_name="s")

    @functools.partial(
        pl.kernel,
        mesh=mesh,
        out_type=jax.ShapeDtypeStruct((n,), jnp.float32),
        compiler_params=pltpu.CompilerParams(needs_layout_passes=False),
        scratch_types=[
            pltpu.VMEM((n,), jnp.float32),
            pltpu.VMEM((n,), jnp.float32),
            pltpu.VMEM((n,), jnp.float32),
            pltpu.VMEM((n + 32,), jnp.float32),
            pltpu.VMEM((rows_per,), jnp.float32),
            pltpu.SemaphoreType.DMA,
            pltpu.SemaphoreType.DMA,
        ],
    )
    def k_sc(x_hbm, o_hbm, row_a, row_b, cost_v, cand_v, out_v, sem0, sem1):
        wid = lax.axis_index("s") * 2 + lax.axis_index("c")
        base = wid * rows_per
        iota = lax.iota(jnp.int32, 16)
        zero16f = jnp.zeros((16,), jnp.float32)
        zero16i = jnp.zeros((16,), jnp.int32)
        sems = (sem0, sem1)
        rows = (row_a, row_b)

        def process(rl, buf):
            rv = rows[buf]
            r = base + rl
            cstart = (r >> 4) << 4
            lane_v = jnp.full((16,), r - cstart, jnp.int32)
            chunk = rv[pl.ds(cstart, 16)]
            d_s = jnp.max(jnp.where(iota == lane_v, chunk, _NEG_INF))
            rv[pl.ds(cstart, 16)] = jnp.where(iota == lane_v, _NEG_INF, chunk)
            md = jnp.full((16,), _MARGIN - d_s, jnp.float32)

            def pa(i, carry):
                m0, m1 = carry
                c0 = i * 64
                t0 = jnp.maximum(rv[pl.ds(c0, 16)] + md, 0.0)
                t1 = jnp.maximum(rv[pl.ds(c0 + 16, 16)] + md, 0.0)
                t2 = jnp.maximum(rv[pl.ds(c0 + 32, 16)] + md, 0.0)
                t3 = jnp.maximum(rv[pl.ds(c0 + 48, 16)] + md, 0.0)
                cost_v[pl.ds(c0, 16)] = t0
                cost_v[pl.ds(c0 + 16, 16)] = t1
                cost_v[pl.ds(c0 + 32, 16)] = t2
                cost_v[pl.ds(c0 + 48, 16)] = t3
                return (jnp.maximum(m0, jnp.maximum(t0, t2)),
                        jnp.maximum(m1, jnp.maximum(t1, t3)))

            m0, m1 = lax.fori_loop(0, n // 64, pa, (zero16f, zero16f))
            tau_s = jnp.min(jnp.minimum(m0, m1))
            rmax_s = jnp.max(jnp.maximum(m0, m1))
            tau_v = jnp.full((16,), tau_s, jnp.float32)

            def pb(g, wp):
                c0 = g * 32
                v0 = cost_v[pl.ds(c0, 16)]
                m0i = (v0 > tau_v).astype(jnp.int32)
                pos0 = plsc.cumsum(m0i) - m0i
                plsc.store_scatter(cand_v, [wp + pos0], v0, mask=v0 > tau_v)
                wp = wp + plsc.all_reduce_population_count(v0 > tau_v)
                v1 = cost_v[pl.ds(c0 + 16, 16)]
                m1i = (v1 > tau_v).astype(jnp.int32)
                pos1 = plsc.cumsum(m1i) - m1i
                plsc.store_scatter(cand_v, [wp + pos1], v1, mask=v1 > tau_v)
                return wp + plsc.all_reduce_population_count(v1 > tau_v)

            wp = lax.fori_loop(0, n // 32, pb, zero16i)
            ls = jnp.max(wp)
            plsc.store_scatter(cand_v, [wp + iota], zero16f)
            nch = (ls + 15) >> 4
            tau_bits = jnp.max(plsc.bitcast(tau_v, jnp.int32))
            rmax_bits = jnp.max(
                plsc.bitcast(jnp.full((16,), rmax_s, jnp.float32), jnp.int32))

            def bcond(ch):
                lo, hi, c_lo = ch
                width = (hi - lo).astype(jnp.float32)
                extra = (c_lo - _K).astype(jnp.float32)
                return width * extra > _TOL

            def bis(ch):
                lo, hi, c_lo = ch
                mid = lo + ((hi - lo + 1) >> 1)
                midv = jnp.full((16,), mid, jnp.int32)

                def cp(c, acc):
                    v = cand_v[pl.ds(c * 16, 16)]
                    kb = plsc.bitcast(v, jnp.int32)
                    return acc + (kb >= midv).astype(jnp.int32)

                acc = lax.fori_loop(0, nch, cp, zero16i)
                cnt = jnp.sum(acc)
                ge = cnt >= _K
                return (jnp.where(ge, mid, lo), jnp.where(ge, hi, mid - 1),
                        jnp.where(ge, cnt, c_lo))

            lo, _hi, _cl = lax.while_loop(
                bcond, bis, (tau_bits, rmax_bits, ls))
            t_bits = jnp.where(ls >= _K, lo, tau_bits)
            tb_v = jnp.full((16,), t_bits, jnp.int32)
            t_val = jnp.max(plsc.bitcast(tb_v, jnp.float32))

            def sp(c, carry):
                s, cc = carry
                v = cand_v[pl.ds(c * 16, 16)]
                kb = plsc.bitcast(v, jnp.int32)
                gt = kb > tb_v
                return s + jnp.where(gt, v, 0.0), cc + gt.astype(jnp.int32)

            s_acc, c_acc = lax.fori_loop(0, nch, sp, (zero16f, zero16i))
            s_row = jnp.sum(s_acc) + (
                (_K - jnp.sum(c_acc)).astype(jnp.float32) * t_val)
            plsc.store_scatter(
                out_v, [jnp.full((16,), rl, jnp.int32)],
                jnp.full((16,), s_row, jnp.float32), mask=iota == 0)

        def dma_start(rl, buf):
            pltpu.make_async_copy(
                x_hbm.at[base + rl], rows[buf], sems[buf]).start()

        def dma_wait(buf):
            pltpu.make_async_copy(
                x_hbm.at[base], rows[buf], sems[buf]).wait()

        dma_start(0, 0)

        def pair(p, _):
            rl0 = p * 2
            dma_start(rl0 + 1, 1)
            dma_wait(0)
            process(rl0, 0)

            @pl.when(rl0 + 2 < rows_per)
            def _():
                dma_start(rl0 + 2, 0)

            dma_wait(1)
            process(rl0 + 1, 1)
            return 0

        lax.fori_loop(0, rows_per // 2, pair, 0)
        pltpu.sync_copy(out_v, o_hbm.at[pl.ds(base, rows_per)])

    return k_sc(scores)


def kernel(scores, neg):
    row = _sc_row_pass(scores)
    col = _run_col_tc(scores, neg)
    return (row + col) / neg


# speedup vs baseline: 1.6822x; 1.5516x over previous
import functools

import jax
import jax.numpy as jnp
from jax import lax
from jax.experimental import pallas as pl
from jax.experimental.pallas import tpu as pltpu
from jax.experimental.pallas import tpu_sc as plsc

_MARGIN = 0.2
_K = 32
_NEG_INF = -3.0e38


_TOL = 4096.0


def _bisect(keys, k, lo, hi, axis, n_iter=None, n_pre=12):
    def step(carry):
        lo, hi, c_lo = carry
        mid = lo + ((hi - lo + 1) >> 1)
        cnt = jnp.sum((keys >= mid).astype(jnp.int32), axis=axis, keepdims=True)
        ge = cnt >= k
        return (jnp.where(ge, mid, lo), jnp.where(ge, hi, mid - 1),
                jnp.where(ge, cnt, c_lo))

    c0 = jnp.full(lo.shape, keys.shape[axis], jnp.int32)
    carry = (lo, hi, c0)
    if n_iter is not None:
        for _ in range(n_iter):
            carry = step(carry)
        lo, hi, _ = carry
    else:
        def cond(c):
            lo, hi, c_lo = c
            width = (hi - lo).astype(jnp.float32)
            extra = (c_lo - k).astype(jnp.float32)
            return jnp.any(width * extra > _TOL)

        for _ in range(n_pre):
            carry = step(carry)
        lo, hi, _ = jax.lax.while_loop(cond, step, carry)
    return lo


def _topk_sum(keys, k, lo, hi, axis):
    t_bits = _bisect(keys, k, lo, hi, axis)
    t = jax.lax.bitcast_convert_type(t_bits, jnp.float32)
    gt = keys > t_bits
    vals = jax.lax.bitcast_convert_type(keys, jnp.float32)
    s = jnp.sum(jnp.where(gt, vals, 0.0), axis=axis, keepdims=True)
    c = jnp.sum(gt.astype(jnp.float32), axis=axis, keepdims=True)
    return s + (k.astype(jnp.float32) - c) * t


def _row_body(neg_ref, x_ref, o_ref, keys_ref, *, block: int, off: int = 0):
    i = pl.program_id(0) + off
    R = block
    x = x_ref[...]
    xd = x_ref[:, pl.ds(i * R, R)]
    rr = jax.lax.broadcasted_iota(jnp.int32, (R, R), 0)
    cc = jax.lax.broadcasted_iota(jnp.int32, (R, R), 1)
    deq = rr == cc
    d = jnp.sum(jnp.where(deq, xd, 0.0), axis=1, keepdims=True)
    cost = jnp.maximum(x + (_MARGIN - d), 0.0)
    keys_ref[...] = jax.lax.bitcast_convert_type(cost, jnp.int32)
    dblk = keys_ref[:, pl.ds(i * R, R)]
    keys_ref[:, pl.ds(i * R, R)] = jnp.where(deq, 0, dblk)
    keys = keys_ref[...]
    k = neg_ref[0]

    m = keys
    w = m.shape[1]
    while w > 128:
        w //= 2
        m = jnp.maximum(m[:, :w], m[:, w:])
    rowmax = jnp.max(m, axis=1, keepdims=True)
    zero = jnp.zeros((R, 1), jnp.int32)
    tau = _bisect(m, k, zero, rowmax, axis=1, n_iter=10)
    o_ref[...] = _topk_sum(keys, k, tau, rowmax, axis=1)


def _col_body(neg_ref, x_ref, o_ref, keys_ref, *, block: int):
    j = pl.program_id(0)
    C = block
    x = x_ref[...]
    xd = x_ref[pl.ds(j * C, C), :]
    rr = jax.lax.broadcasted_iota(jnp.int32, (C, C), 0)
    cc = jax.lax.broadcasted_iota(jnp.int32, (C, C), 1)
    deq = rr == cc
    d = jnp.sum(jnp.where(deq, xd, 0.0), axis=0, keepdims=True)
    cost = jnp.maximum(x + (_MARGIN - d), 0.0)
    keys_ref[...] = jax.lax.bitcast_convert_type(cost, jnp.int32)
    dblk = keys_ref[pl.ds(j * C, C), :]
    keys_ref[pl.ds(j * C, C), :] = jnp.where(deq, 0, dblk)
    keys = keys_ref[...]
    k = neg_ref[0]

    m = keys
    w = m.shape[0]
    while w > 128:
        w //= 2
        m = jnp.maximum(m[:w, :], m[w:, :])
    colmax = jnp.max(m, axis=0, keepdims=True)
    zero = jnp.zeros((1, C), jnp.int32)
    tau = _bisect(m, k, zero, colmax, axis=0, n_iter=10)
    res = _topk_sum(keys, k, tau, colmax, axis=0)
    o_ref[...] = jnp.broadcast_to(res, o_ref.shape)


def _run(scores, neg, *, block: int = 256, interpret: bool = False):
    n = scores.shape[0]
    neg_arr = jnp.asarray(neg, jnp.int32).reshape(1)
    grid = (n // block,)

    row_out = pl.pallas_call(
        functools.partial(_row_body, block=block),
        grid=grid,
        in_specs=[
            pl.BlockSpec(memory_space=pltpu.SMEM),
            pl.BlockSpec((block, n), lambda i: (i, 0)),
        ],
        out_specs=pl.BlockSpec((block, 1), lambda i: (i, 0)),
        out_shape=jax.ShapeDtypeStruct((n, 1), jnp.float32),
        scratch_shapes=[pltpu.VMEM((block, n), jnp.int32)],
        interpret=interpret,
    )(neg_arr, scores)

    col_out = pl.pallas_call(
        functools.partial(_col_body, block=block),
        grid=grid,
        in_specs=[
            pl.BlockSpec(memory_space=pltpu.SMEM),
            pl.BlockSpec((n, block), lambda j: (0, j)),
        ],
        out_specs=pl.BlockSpec((8, block), lambda j: (0, j)),
        out_shape=jax.ShapeDtypeStruct((8, n), jnp.float32),
        scratch_shapes=[pltpu.VMEM((n, block), jnp.int32)],
        interpret=interpret,
    )(neg_arr, scores)

    return (row_out[:, 0] + col_out[0, :]) / neg


def _run_row_tc(scores, neg, row0, *, block: int = 256):
    n = scores.shape[0]
    neg_arr = jnp.asarray(neg, jnp.int32).reshape(1)
    off = row0 // block
    row_out = pl.pallas_call(
        functools.partial(_row_body, block=block, off=off),
        grid=((n - row0) // block,),
        in_specs=[
            pl.BlockSpec(memory_space=pltpu.SMEM),
            pl.BlockSpec((block, n), lambda i: (i + off, 0)),
        ],
        out_specs=pl.BlockSpec((block, 1), lambda i: (i, 0)),
        out_shape=jax.ShapeDtypeStruct((n - row0, 1), jnp.float32),
        scratch_shapes=[pltpu.VMEM((block, n), jnp.int32)],
    )(neg_arr, scores)
    return row_out[:, 0]


def _run_col_tc(scores, neg, *, block: int = 256):
    n = scores.shape[0]
    neg_arr = jnp.asarray(neg, jnp.int32).reshape(1)
    col_out = pl.pallas_call(
        functools.partial(_col_body, block=block),
        grid=(n // block,),
        in_specs=[
            pl.BlockSpec(memory_space=pltpu.SMEM),
            pl.BlockSpec((n, block), lambda j: (0, j)),
        ],
        out_specs=pl.BlockSpec((8, block), lambda j: (0, j)),
        out_shape=jax.ShapeDtypeStruct((8, n), jnp.float32),
        scratch_shapes=[pltpu.VMEM((n, block), jnp.int32)],
    )(neg_arr, scores)
    return col_out[0, :]


def _sc_row_pass(scores, nrows):
    n = scores.shape[0]
    nw = 32
    rows_per = nrows // nw
    mesh = plsc.VectorSubcoreMesh(core_axis_name="c", subcore_axis_name="s")

    @functools.partial(
        pl.kernel,
        mesh=mesh,
        out_type=jax.ShapeDtypeStruct((nrows,), jnp.float32),
        compiler_params=pltpu.CompilerParams(needs_layout_passes=False),
        scratch_types=[
            pltpu.VMEM((n,), jnp.float32),
            pltpu.VMEM((n,), jnp.float32),
            pltpu.VMEM((n,), jnp.float32),
            pltpu.VMEM((n + 32,), jnp.float32),
            pltpu.VMEM((rows_per,), jnp.float32),
            pltpu.SemaphoreType.DMA,
            pltpu.SemaphoreType.DMA,
        ],
    )
    def k_sc(x_hbm, o_hbm, row_a, row_b, cost_v, cand_v, out_v, sem0, sem1):
        wid = lax.axis_index("s") * 2 + lax.axis_index("c")
        base = wid * rows_per
        iota = lax.iota(jnp.int32, 16)
        zero16f = jnp.zeros((16,), jnp.float32)
        zero16i = jnp.zeros((16,), jnp.int32)
        sems = (sem0, sem1)
        rows = (row_a, row_b)

        def process(rl, buf):
            rv = rows[buf]
            r = base + rl
            cstart = (r >> 4) << 4
            lane_v = jnp.full((16,), r - cstart, jnp.int32)
            chunk = rv[pl.ds(cstart, 16)]
            d_s = jnp.max(jnp.where(iota == lane_v, chunk, _NEG_INF))
            rv[pl.ds(cstart, 16)] = jnp.where(iota == lane_v, _NEG_INF, chunk)
            md = jnp.full((16,), _MARGIN - d_s, jnp.float32)

            def pa(i, carry):
                m0, m1 = carry
                c0 = i * 64
                t0 = jnp.maximum(rv[pl.ds(c0, 16)] + md, 0.0)
                t1 = jnp.maximum(rv[pl.ds(c0 + 16, 16)] + md, 0.0)
                t2 = jnp.maximum(rv[pl.ds(c0 + 32, 16)] + md, 0.0)
                t3 = jnp.maximum(rv[pl.ds(c0 + 48, 16)] + md, 0.0)
                cost_v[pl.ds(c0, 16)] = t0
                cost_v[pl.ds(c0 + 16, 16)] = t1
                cost_v[pl.ds(c0 + 32, 16)] = t2
                cost_v[pl.ds(c0 + 48, 16)] = t3
                return (jnp.maximum(m0, jnp.maximum(t0, t2)),
                        jnp.maximum(m1, jnp.maximum(t1, t3)))

            m0, m1 = lax.fori_loop(0, n // 64, pa, (zero16f, zero16f))
            tau_s = jnp.min(jnp.minimum(m0, m1))
            rmax_s = jnp.max(jnp.maximum(m0, m1))
            tau_v = jnp.full((16,), tau_s, jnp.float32)

            def pb(g, wp):
                c0 = g * 32
                v0 = cost_v[pl.ds(c0, 16)]
                m0i = (v0 > tau_v).astype(jnp.int32)
                pos0 = plsc.cumsum(m0i) - m0i
                plsc.store_scatter(cand_v, [wp + pos0], v0, mask=v0 > tau_v)
                wp = wp + plsc.all_reduce_population_count(v0 > tau_v)
                v1 = cost_v[pl.ds(c0 + 16, 16)]
                m1i = (v1 > tau_v).astype(jnp.int32)
                pos1 = plsc.cumsum(m1i) - m1i
                plsc.store_scatter(cand_v, [wp + pos1], v1, mask=v1 > tau_v)
                return wp + plsc.all_reduce_population_count(v1 > tau_v)

            wp = lax.fori_loop(0, n // 32, pb, zero16i)
            ls = jnp.max(wp)
            plsc.store_scatter(cand_v, [wp + iota], zero16f)
            nch = (ls + 15) >> 4
            tau_bits = jnp.max(plsc.bitcast(tau_v, jnp.int32))
            rmax_bits = jnp.max(
                plsc.bitcast(jnp.full((16,), rmax_s, jnp.float32), jnp.int32))

            def bcond(ch):
                lo, hi, c_lo = ch
                width = (hi - lo).astype(jnp.float32)
                extra = (c_lo - _K).astype(jnp.float32)
                return width * extra > _TOL

            def bis(ch):
                lo, hi, c_lo = ch
                mid = lo + ((hi - lo + 1) >> 1)
                midv = jnp.full((16,), mid, jnp.int32)

                def cp(c, acc):
                    v = cand_v[pl.ds(c * 16, 16)]
                    kb = plsc.bitcast(v, jnp.int32)
                    return acc + (kb >= midv).astype(jnp.int32)

                acc = lax.fori_loop(0, nch, cp, zero16i)
                cnt = jnp.sum(acc)
                ge = cnt >= _K
                return (jnp.where(ge, mid, lo), jnp.where(ge, hi, mid - 1),
                        jnp.where(ge, cnt, c_lo))

            lo, _hi, _cl = lax.while_loop(
                bcond, bis, (tau_bits, rmax_bits, ls))
            t_bits = jnp.where(ls >= _K, lo, tau_bits)
            tb_v = jnp.full((16,), t_bits, jnp.int32)
            t_val = jnp.max(plsc.bitcast(tb_v, jnp.float32))

            def sp(c, carry):
                s, cc = carry
                v = cand_v[pl.ds(c * 16, 16)]
                kb = plsc.bitcast(v, jnp.int32)
                gt = kb > tb_v
                return s + jnp.where(gt, v, 0.0), cc + gt.astype(jnp.int32)

            s_acc, c_acc = lax.fori_loop(0, nch, sp, (zero16f, zero16i))
            s_row = jnp.sum(s_acc) + (
                (_K - jnp.sum(c_acc)).astype(jnp.float32) * t_val)
            plsc.store_scatter(
                out_v, [jnp.full((16,), rl, jnp.int32)],
                jnp.full((16,), s_row, jnp.float32), mask=iota == 0)

        def dma_start(rl, buf):
            pltpu.make_async_copy(
                x_hbm.at[base + rl], rows[buf], sems[buf]).start()

        def dma_wait(buf):
            pltpu.make_async_copy(
                x_hbm.at[base], rows[buf], sems[buf]).wait()

        dma_start(0, 0)

        def pair(p, _):
            rl0 = p * 2
            dma_start(rl0 + 1, 1)
            dma_wait(0)
            process(rl0, 0)

            @pl.when(rl0 + 2 < rows_per)
            def _():
                dma_start(rl0 + 2, 0)

            dma_wait(1)
            process(rl0 + 1, 1)
            return 0

        lax.fori_loop(0, rows_per // 2, pair, 0)
        pltpu.sync_copy(out_v, o_hbm.at[pl.ds(base, rows_per)])

    return k_sc(scores)


_SC_ROWS = 5120


def kernel(scores, neg):
    row_sc = _sc_row_pass(scores, _SC_ROWS)
    row_tc = _run_row_tc(scores, neg, _SC_ROWS)
    col = _run_col_tc(scores, neg)
    row = jnp.concatenate([row_sc, row_tc])
    return (row + col) / neg
